# baseline (speedup 1.0000x reference)
import functools

import jax
import jax.numpy as jnp
from jax import lax
from jax.experimental import pallas as pl
from jax.experimental.pallas import tpu as pltpu
from jax.experimental.pallas import tpu_sc as plsc

_B = 1024
_S = 4096
_K = 4
_E = 16
_L = 16
_NW = 32
_BPW = _B // _NW
_CH = 2048
_NCH = _S // _CH


def _sc_kernel(x_hbm, ebc_hbm, p_hbm, o_hbm, p_v, x_v, o_v, e_v, sem):
    wid = lax.axis_index("s") * 2 + lax.axis_index("c")
    pltpu.sync_copy(ebc_hbm, e_v)
    for c in range(_NCH):
        pltpu.sync_copy(p_hbm.at[:, pl.ds(c * _CH, _CH)], p_v)

        def run_batch(i, carry):
            b = wid * _BPW + i
            pltpu.sync_copy(x_hbm.at[b, :, pl.ds(c * _CH, _CH)], x_v)
            for eh in range(2):
                ek = [[e_v[k, eh * 8 + e] for k in range(_K)] for e in range(8)]

                def inner(j, carry2):
                    xv = [x_v[k, pl.ds(j * _L, _L)] for k in range(_K)]
                    for e in range(8):
                        acc = p_v[eh * 8 + e, pl.ds(j * _L, _L)]
                        for k in range(_K):
                            acc = acc + xv[k] * ek[e][k]
                        o_v[eh * 8 + e, pl.ds(j * _L, _L)] = acc
                    return carry2

                lax.fori_loop(0, _CH // _L, inner, 0)
            pltpu.sync_copy(o_v, o_hbm.at[b, :, pl.ds(c * _CH, _CH)])
            return carry

        lax.fori_loop(0, _BPW, run_batch, 0)


def kernel(inputs, embedding, pos_table):
    xt = jnp.transpose(inputs, (0, 2, 1))
    pt = jnp.transpose(pos_table, (1, 0))
    ebc = jnp.broadcast_to(embedding[:, :, None], (_K, _E, _L))

    mesh = plsc.VectorSubcoreMesh(core_axis_name="c", subcore_axis_name="s")
    run = functools.partial(
        pl.kernel,
        mesh=mesh,
        out_type=jax.ShapeDtypeStruct((_B, _E, _S), jnp.float32),
        scratch_types=[
            pltpu.VMEM((_E, _CH), jnp.float32),
            pltpu.VMEM((_K, _CH), jnp.float32),
            pltpu.VMEM((_E, _CH), jnp.float32),
            pltpu.VMEM((_K, _E, _L), jnp.float32),
            pltpu.SemaphoreType.DMA,
        ],
    )(_sc_kernel)
    out_t = run(xt, ebc, pt)
    return jnp.transpose(out_t, (0, 2, 1))
